# Initial kernel scaffold; baseline (speedup 1.0000x reference)
import functools
import jax
import jax.numpy as jnp
from jax import lax
from jax.experimental import pallas as pl
from jax.experimental.pallas import tpu as pltpu
from jax.experimental.pallas import tpu_sc as plsc

N = 10000
E = 320000
D_IN = 128
H = 64
C = 40

NC = 2
NS = 16
NW = NC * NS
K = 80
CHUNKS = E // K
CW = CHUNKS // NW
ROWS_PER_TILE = N // NS

_mesh = plsc.VectorSubcoreMesh(core_axis_name="c", subcore_axis_name="s")


def _deg_body(src_hbm, dst_hbm, zeros_hbm, ones_hbm, out_hbm,
              idx_s, idx_d, ones_v, deg_s, deg_d):
    c = lax.axis_index("c")
    s = lax.axis_index("s")
    w = c * NS + s

    pltpu.sync_copy(ones_hbm, ones_v)
    pltpu.sync_copy(src_hbm.at[pl.ds(w * CW, CW)], idx_s)
    pltpu.sync_copy(dst_hbm.at[pl.ds(w * CW, CW)], idx_d)

    rbase = s * ROWS_PER_TILE
    pltpu.sync_copy(zeros_hbm.at[pl.ds(rbase, ROWS_PER_TILE)],
                    deg_s.at[pl.ds(rbase, ROWS_PER_TILE)])
    pltpu.sync_copy(zeros_hbm.at[pl.ds(rbase, ROWS_PER_TILE)],
                    deg_d.at[pl.ds(rbase, ROWS_PER_TILE)])
    plsc.subcore_barrier()

    def body(j, carry):
        pltpu.sync_copy(ones_v, deg_s.at[idx_s.at[j]], add=True)
        pltpu.sync_copy(ones_v, deg_d.at[idx_d.at[j]], add=True)
        return carry

    lax.fori_loop(0, CW, body, 0)
    plsc.subcore_barrier()

    pltpu.sync_copy(deg_s.at[pl.ds(rbase, ROWS_PER_TILE)],
                    out_hbm.at[c, 0, pl.ds(rbase, ROWS_PER_TILE)])
    pltpu.sync_copy(deg_d.at[pl.ds(rbase, ROWS_PER_TILE)],
                    out_hbm.at[c, 1, pl.ds(rbase, ROWS_PER_TILE)])


_sc_degrees = pl.kernel(
    _deg_body,
    out_type=jax.ShapeDtypeStruct((NC, 2, N, 16), jnp.float32),
    mesh=_mesh,
    scratch_types=[
        pltpu.VMEM((CW, K), jnp.int32),
        pltpu.VMEM((CW, K), jnp.int32),
        pltpu.VMEM((K, 16), jnp.float32),
        pltpu.VMEM_SHARED((N, 16), jnp.float32),
        pltpu.VMEM_SHARED((N, 16), jnp.float32),
    ],
)


def _layer_body(hw_hbm, src_hbm, dst_hbm, zeros_hbm, out_hbm,
                idx_s, idx_d, msgs, agg_sh, sem):
    c = lax.axis_index("c")
    s = lax.axis_index("s")
    w = c * NS + s

    pltpu.sync_copy(src_hbm.at[pl.ds(w * CW, CW)], idx_s)
    pltpu.sync_copy(dst_hbm.at[pl.ds(w * CW, CW)], idx_d)

    rbase = s * ROWS_PER_TILE
    pltpu.sync_copy(zeros_hbm.at[pl.ds(rbase, ROWS_PER_TILE)],
                    agg_sh.at[pl.ds(rbase, ROWS_PER_TILE)])
    plsc.subcore_barrier()

    def body(j, carry):
        pltpu.async_copy(hw_hbm.at[idx_s.at[j]], msgs, sem).wait()
        pltpu.sync_copy(msgs, agg_sh.at[idx_d.at[j]], add=True)
        return carry

    lax.fori_loop(0, CW, body, 0)
    plsc.subcore_barrier()

    pltpu.sync_copy(agg_sh.at[pl.ds(rbase, ROWS_PER_TILE)],
                    out_hbm.at[c, pl.ds(rbase, ROWS_PER_TILE)])


_sc_layer = pl.kernel(
    _layer_body,
    out_type=jax.ShapeDtypeStruct((NC, N, H), jnp.float32),
    mesh=_mesh,
    scratch_types=[
        pltpu.VMEM((CW, K), jnp.int32),
        pltpu.VMEM((CW, K), jnp.int32),
        pltpu.VMEM((K, H), jnp.float32),
        pltpu.VMEM_SHARED((N, H), jnp.float32),
        pltpu.SemaphoreType.DMA,
    ],
)


_B = 1000
_GRID = N // _B


def _norms_body(degs_ref, out_ref):
    d = degs_ref[...]
    dsrc = d[0, 0] + d[1, 0]
    ddst = d[0, 1] + d[1, 1]
    nsrc = lax.rsqrt(jnp.maximum(dsrc[:, :1], 1.0))
    ndst = lax.rsqrt(jnp.maximum(ddst[:, :1], 1.0))
    out_ref[0] = jnp.broadcast_to(nsrc, (_B, H))
    out_ref[1] = jnp.broadcast_to(ndst, (_B, H))


def _tc_norms(degs):
    return pl.pallas_call(
        _norms_body,
        grid=(_GRID,),
        in_specs=[pl.BlockSpec((NC, 2, _B, 16), lambda i: (0, 0, i, 0))],
        out_specs=pl.BlockSpec((2, _B, H), lambda i: (0, i, 0)),
        out_shape=jax.ShapeDtypeStruct((2, N, H), jnp.float32),
    )(degs)


def _first_body(x_ref, norms_ref, w_ref, out_ref):
    xs = x_ref[...] * norms_ref[0]
    out_ref[...] = jnp.dot(xs, w_ref[...], preferred_element_type=jnp.float32)


def _tc_first(x, norms, W0):
    return pl.pallas_call(
        _first_body,
        grid=(_GRID,),
        in_specs=[
            pl.BlockSpec((_B, D_IN), lambda i: (i, 0)),
            pl.BlockSpec((2, _B, H), lambda i: (0, i, 0)),
            pl.BlockSpec((D_IN, H), lambda i: (0, 0)),
        ],
        out_specs=pl.BlockSpec((_B, H), lambda i: (i, 0)),
        out_shape=jax.ShapeDtypeStruct((N, H), jnp.float32),
    )(x, norms, W0)


def _mid_body(agg_ref, norms_ref, b_ref, w_ref, out_ref):
    agg = agg_ref[0] + agg_ref[1]
    h = agg * norms_ref[1] + b_ref[...]
    h = jnp.where(h >= 0.0, h, 0.01 * h)
    out_ref[...] = jnp.dot(h * norms_ref[0], w_ref[...],
                           preferred_element_type=jnp.float32)


def _tc_mid(agg, norms, b, W):
    return pl.pallas_call(
        _mid_body,
        grid=(_GRID,),
        in_specs=[
            pl.BlockSpec((NC, _B, H), lambda i: (0, i, 0)),
            pl.BlockSpec((2, _B, H), lambda i: (0, i, 0)),
            pl.BlockSpec((1, H), lambda i: (0, 0)),
            pl.BlockSpec((H, H), lambda i: (0, 0)),
        ],
        out_specs=pl.BlockSpec((_B, H), lambda i: (i, 0)),
        out_shape=jax.ShapeDtypeStruct((N, H), jnp.float32),
    )(agg, norms, b, W)


def _final_body(agg_ref, norms_ref, b_ref, fcw_ref, fcb_ref, out_ref):
    agg = agg_ref[0] + agg_ref[1]
    h = agg * norms_ref[1] + b_ref[...]
    h = jnp.where(h >= 0.0, h, 0.01 * h)
    out_ref[...] = (jnp.dot(h, fcw_ref[...], preferred_element_type=jnp.float32)
                    + fcb_ref[...])


def _tc_final(agg, norms, b, fcW, fcb):
    return pl.pallas_call(
        _final_body,
        grid=(_GRID,),
        in_specs=[
            pl.BlockSpec((NC, _B, H), lambda i: (0, i, 0)),
            pl.BlockSpec((2, _B, H), lambda i: (0, i, 0)),
            pl.BlockSpec((1, H), lambda i: (0, 0)),
            pl.BlockSpec((H, C), lambda i: (0, 0)),
            pl.BlockSpec((1, C), lambda i: (0, 0)),
        ],
        out_specs=pl.BlockSpec((_B, C), lambda i: (i, 0)),
        out_shape=jax.ShapeDtypeStruct((N, C), jnp.float32),
    )(agg, norms, b, fcW, fcb)


def kernel(x, edge_index, Ws, bs, fcW, fcb):
    src2d = edge_index[0].reshape(CHUNKS, K)
    dst2d = edge_index[1].reshape(CHUNKS, K)
    zeros16 = jnp.zeros((N, 16), jnp.float32)
    zeros64 = jnp.zeros((N, H), jnp.float32)
    ones80 = jnp.ones((K, 16), jnp.float32)

    degs = _sc_degrees(src2d, dst2d, zeros16, ones80)
    norms = _tc_norms(degs)

    hw = _tc_first(x, norms, Ws[0])
    for l in range(1, len(Ws)):
        agg = _sc_layer(hw, src2d, dst2d, zeros64)
        hw = _tc_mid(agg, norms, bs[l - 1].reshape(1, H), Ws[l])
    agg = _sc_layer(hw, src2d, dst2d, zeros64)
    return _tc_final(agg, norms, bs[-1].reshape(1, H), fcW, fcb)


# speedup vs baseline: 7.9435x; 7.9435x over previous
import functools
import jax
import jax.numpy as jnp
from jax import lax
from jax.experimental import pallas as pl
from jax.experimental.pallas import tpu as pltpu
from jax.experimental.pallas import tpu_sc as plsc

N = 10000
E = 320000
D_IN = 128
H = 64
C = 40

NC = 2
NS = 16
NW = NC * NS
K = 80
CHUNKS = E // K
CW = CHUNKS // NW
RPT = 640
NPAD = NS * RPT


def _deg_body(src_hbm, dst_hbm, zeros_hbm, ones_hbm, out_hbm,
              idx_s, idx_d, ones_v, deg_s, deg_d):
    c = lax.axis_index("c")
    s = lax.axis_index("s")
    w = c * NS + s

    pltpu.sync_copy(ones_hbm, ones_v)
    pltpu.sync_copy(src_hbm.at[w], idx_s)
    pltpu.sync_copy(dst_hbm.at[w], idx_d)

    rbase = s * RPT
    pltpu.sync_copy(zeros_hbm.at[s], deg_s.at[pl.ds(rbase, RPT)])
    pltpu.sync_copy(zeros_hbm.at[s], deg_d.at[pl.ds(rbase, RPT)])
    plsc.subcore_barrier()

    def body(j, carry):
        pltpu.sync_copy(ones_v, deg_s.at[idx_s.at[j]], add=True)
        pltpu.sync_copy(ones_v, deg_d.at[idx_d.at[j]], add=True)
        return carry

    lax.fori_loop(0, CW, body, 0)
    plsc.subcore_barrier()

    pltpu.sync_copy(deg_s.at[pl.ds(rbase, RPT)], out_hbm.at[c, 0, s])
    pltpu.sync_copy(deg_d.at[pl.ds(rbase, RPT)], out_hbm.at[c, 1, s])


@functools.lru_cache(maxsize=None)
def _sc_degrees_kernel():
    mesh = plsc.VectorSubcoreMesh(core_axis_name="c", subcore_axis_name="s",
                                  num_cores=NC, num_subcores=NS)
    return pl.kernel(
        _deg_body,
        out_type=jax.ShapeDtypeStruct((NC, 2, NS, RPT, 16), jnp.float32),
        mesh=mesh,
        compiler_params=pltpu.CompilerParams(use_tc_tiling_on_sc=False),
        scratch_types=[
            pltpu.VMEM((CW, K), jnp.int32),
            pltpu.VMEM((CW, K), jnp.int32),
            pltpu.VMEM((K, 16), jnp.float32),
            pltpu.VMEM_SHARED((NPAD, 16), jnp.float32),
            pltpu.VMEM_SHARED((NPAD, 16), jnp.float32),
        ],
    )


def _layer_body(hw_hbm, src_hbm, dst_hbm, zeros_hbm, out_hbm,
                idx_s, idx_d, msgs, agg_sh, sem):
    c = lax.axis_index("c")
    s = lax.axis_index("s")
    w = c * NS + s

    pltpu.sync_copy(src_hbm.at[w], idx_s)
    pltpu.sync_copy(dst_hbm.at[w], idx_d)

    rbase = s * RPT
    pltpu.sync_copy(zeros_hbm.at[s], agg_sh.at[pl.ds(rbase, RPT)])
    plsc.subcore_barrier()

    def body(j, carry):
        pltpu.async_copy(hw_hbm.at[idx_s.at[j]], msgs, sem).wait()
        pltpu.sync_copy(msgs, agg_sh.at[idx_d.at[j]], add=True)
        return carry

    lax.fori_loop(0, CW, body, 0)
    plsc.subcore_barrier()

    pltpu.sync_copy(agg_sh.at[pl.ds(rbase, RPT)], out_hbm.at[c, s])


@functools.lru_cache(maxsize=None)
def _sc_layer_kernel():
    mesh = plsc.VectorSubcoreMesh(core_axis_name="c", subcore_axis_name="s",
                                  num_cores=NC, num_subcores=NS)
    return pl.kernel(
        _layer_body,
        out_type=jax.ShapeDtypeStruct((NC, NS, RPT, H), jnp.float32),
        mesh=mesh,
        compiler_params=pltpu.CompilerParams(use_tc_tiling_on_sc=False),
        scratch_types=[
            pltpu.VMEM((CW, K), jnp.int32),
            pltpu.VMEM((CW, K), jnp.int32),
            pltpu.VMEM((K, H), jnp.float32),
            pltpu.VMEM_SHARED((NPAD, H), jnp.float32),
            pltpu.SemaphoreType.DMA,
        ],
    )


_B = 1000
_GRID = N // _B


def _norms_body(degs_ref, out_ref):
    d = degs_ref[...]
    dsrc = d[0, 0] + d[1, 0]
    ddst = d[0, 1] + d[1, 1]
    nsrc = lax.rsqrt(jnp.maximum(dsrc[:, :1], 1.0))
    ndst = lax.rsqrt(jnp.maximum(ddst[:, :1], 1.0))
    out_ref[0] = jnp.broadcast_to(nsrc, (_B, H))
    out_ref[1] = jnp.broadcast_to(ndst, (_B, H))


def _tc_norms(degs):
    return pl.pallas_call(
        _norms_body,
        grid=(_GRID,),
        in_specs=[pl.BlockSpec((NC, 2, _B, 16), lambda i: (0, 0, i, 0))],
        out_specs=pl.BlockSpec((2, _B, H), lambda i: (0, i, 0)),
        out_shape=jax.ShapeDtypeStruct((2, N, H), jnp.float32),
    )(degs)


def _first_body(x_ref, norms_ref, w_ref, out_ref):
    out_ref[...] = jnp.dot(x_ref[...], w_ref[...],
                           preferred_element_type=jnp.float32) * norms_ref[0]


def _tc_first(x, norms, W0):
    return pl.pallas_call(
        _first_body,
        grid=(_GRID,),
        in_specs=[
            pl.BlockSpec((_B, D_IN), lambda i: (i, 0)),
            pl.BlockSpec((2, _B, H), lambda i: (0, i, 0)),
            pl.BlockSpec((D_IN, H), lambda i: (0, 0)),
        ],
        out_specs=pl.BlockSpec((_B, H), lambda i: (i, 0)),
        out_shape=jax.ShapeDtypeStruct((N, H), jnp.float32),
    )(x, norms, W0)


def _mid_body(agg_ref, norms_ref, b_ref, w_ref, out_ref):
    agg = agg_ref[0] + agg_ref[1]
    h = agg * norms_ref[1] + b_ref[...]
    h = jnp.where(h >= 0.0, h, 0.01 * h)
    out_ref[...] = jnp.dot(h * norms_ref[0], w_ref[...],
                           preferred_element_type=jnp.float32)


def _tc_mid(agg, norms, b, W):
    return pl.pallas_call(
        _mid_body,
        grid=(_GRID,),
        in_specs=[
            pl.BlockSpec((NC, _B, H), lambda i: (0, i, 0)),
            pl.BlockSpec((2, _B, H), lambda i: (0, i, 0)),
            pl.BlockSpec((1, H), lambda i: (0, 0)),
            pl.BlockSpec((H, H), lambda i: (0, 0)),
        ],
        out_specs=pl.BlockSpec((_B, H), lambda i: (i, 0)),
        out_shape=jax.ShapeDtypeStruct((N, H), jnp.float32),
    )(agg, norms, b, W)


def _final_body(agg_ref, norms_ref, b_ref, fcw_ref, fcb_ref, out_ref):
    agg = agg_ref[0] + agg_ref[1]
    h = agg * norms_ref[1] + b_ref[...]
    h = jnp.where(h >= 0.0, h, 0.01 * h)
    out_ref[...] = (jnp.dot(h, fcw_ref[...], preferred_element_type=jnp.float32)
                    + fcb_ref[...])


def _tc_final(agg, norms, b, fcW, fcb):
    return pl.pallas_call(
        _final_body,
        grid=(_GRID,),
        in_specs=[
            pl.BlockSpec((NC, _B, H), lambda i: (0, i, 0)),
            pl.BlockSpec((2, _B, H), lambda i: (0, i, 0)),
            pl.BlockSpec((1, H), lambda i: (0, 0)),
            pl.BlockSpec((H, C), lambda i: (0, 0)),
            pl.BlockSpec((1, C), lambda i: (0, 0)),
        ],
        out_specs=pl.BlockSpec((_B, C), lambda i: (i, 0)),
        out_shape=jax.ShapeDtypeStruct((N, C), jnp.float32),
    )(agg, norms, b, fcW, fcb)


def kernel(x, edge_index, Ws, bs, fcW, fcb):
    src3d = edge_index[0].reshape(NW, CW, K)
    dst3d = edge_index[1].reshape(NW, CW, K)
    zeros16 = jnp.zeros((NS, RPT, 16), jnp.float32)
    zeros64 = jnp.zeros((NS, RPT, H), jnp.float32)
    ones80 = jnp.ones((K, 16), jnp.float32)

    sc_degrees = _sc_degrees_kernel()
    sc_layer = _sc_layer_kernel()

    degs = sc_degrees(src3d, dst3d, zeros16, ones80)
    degs = degs.reshape(NC, 2, NPAD, 16)[:, :, :N]
    norms = _tc_norms(degs)

    hw = _tc_first(x, norms, Ws[0])
    for l in range(1, len(Ws)):
        agg = sc_layer(hw, src3d, dst3d, zeros64)
        agg = agg.reshape(NC, NPAD, H)[:, :N]
        hw = _tc_mid(agg, norms, bs[l - 1].reshape(1, H), Ws[l])
    agg = sc_layer(hw, src3d, dst3d, zeros64)
    agg = agg.reshape(NC, NPAD, H)[:, :N]
    return _tc_final(agg, norms, bs[-1].reshape(1, H), fcW, fcb.reshape(1, C))


# speedup vs baseline: 14.4317x; 1.8168x over previous
import functools
import jax
import jax.numpy as jnp
from jax import lax
from jax.experimental import pallas as pl
from jax.experimental.pallas import tpu as pltpu
from jax.experimental.pallas import tpu_sc as plsc

N = 10000
E = 320000
D_IN = 128
H = 64
C = 40

NC = 2
NS = 16
NW = NC * NS
K = 80
CHUNKS = E // K
CW = CHUNKS // NW
RPT = 640
NPAD = NS * RPT


def _deg_body(src_hbm, dst_hbm, zeros_hbm, ones_hbm, out_hbm,
              idx_s, idx_d, ones_v, deg_s, deg_d):
    c = lax.axis_index("c")
    s = lax.axis_index("s")
    w = c * NS + s

    pltpu.sync_copy(ones_hbm, ones_v)
    pltpu.sync_copy(src_hbm.at[w], idx_s)
    pltpu.sync_copy(dst_hbm.at[w], idx_d)

    rbase = s * RPT
    pltpu.sync_copy(zeros_hbm.at[s], deg_s.at[pl.ds(rbase, RPT)])
    pltpu.sync_copy(zeros_hbm.at[s], deg_d.at[pl.ds(rbase, RPT)])
    plsc.subcore_barrier()

    def body(j, carry):
        pltpu.sync_copy(ones_v, deg_s.at[idx_s.at[j]], add=True)
        pltpu.sync_copy(ones_v, deg_d.at[idx_d.at[j]], add=True)
        return carry

    lax.fori_loop(0, CW, body, 0)
    plsc.subcore_barrier()

    pltpu.sync_copy(deg_s.at[pl.ds(rbase, RPT)], out_hbm.at[c, 0, s])
    pltpu.sync_copy(deg_d.at[pl.ds(rbase, RPT)], out_hbm.at[c, 1, s])


@functools.lru_cache(maxsize=None)
def _sc_degrees_kernel():
    mesh = plsc.VectorSubcoreMesh(core_axis_name="c", subcore_axis_name="s",
                                  num_cores=NC, num_subcores=NS)
    return pl.kernel(
        _deg_body,
        out_type=jax.ShapeDtypeStruct((NC, 2, NS, RPT, 16), jnp.float32),
        mesh=mesh,
        compiler_params=pltpu.CompilerParams(use_tc_tiling_on_sc=False),
        scratch_types=[
            pltpu.VMEM((CW, K), jnp.int32),
            pltpu.VMEM((CW, K), jnp.int32),
            pltpu.VMEM((K, 16), jnp.float32),
            pltpu.VMEM_SHARED((NPAD, 16), jnp.float32),
            pltpu.VMEM_SHARED((NPAD, 16), jnp.float32),
        ],
    )


NB = 5
NG = CW // NB


def _layer_body(hw_hbm, src_hbm, dst_hbm, zeros_hbm, out_hbm,
                idx_s, idx_d, msgs, agg_sh, gsems, ssems):
    c = lax.axis_index("c")
    s = lax.axis_index("s")
    w = c * NS + s

    pltpu.sync_copy(src_hbm.at[w], idx_s)
    pltpu.sync_copy(dst_hbm.at[w], idx_d)

    rbase = s * RPT
    pltpu.sync_copy(zeros_hbm.at[s], agg_sh.at[pl.ds(rbase, RPT)])
    plsc.subcore_barrier()

    dummy = hw_hbm.at[pl.ds(0, K)]

    for b in range(NB):
        pltpu.async_copy(hw_hbm.at[idx_s.at[b]], msgs.at[b], gsems.at[b])

    def group(g, carry):
        for b in range(NB):
            pltpu.make_async_copy(dummy, msgs.at[b], gsems.at[b]).wait()
            pltpu.async_copy(msgs.at[b], agg_sh.at[idx_d.at[g * NB + b]],
                             ssems.at[b], add=True)

        @pl.when(g < NG - 1)
        def _():
            for b in range(NB):
                pltpu.make_async_copy(dummy, msgs.at[b], ssems.at[b]).wait()
                pltpu.async_copy(hw_hbm.at[idx_s.at[(g + 1) * NB + b]],
                                 msgs.at[b], gsems.at[b])
        return carry

    lax.fori_loop(0, NG, group, 0)
    for b in range(NB):
        pltpu.make_async_copy(dummy, msgs.at[b], ssems.at[b]).wait()
    plsc.subcore_barrier()

    pltpu.sync_copy(agg_sh.at[pl.ds(rbase, RPT)], out_hbm.at[c, s])


@functools.lru_cache(maxsize=None)
def _sc_layer_kernel():
    mesh = plsc.VectorSubcoreMesh(core_axis_name="c", subcore_axis_name="s",
                                  num_cores=NC, num_subcores=NS)
    return pl.kernel(
        _layer_body,
        out_type=jax.ShapeDtypeStruct((NC, NS, RPT, H), jnp.float32),
        mesh=mesh,
        compiler_params=pltpu.CompilerParams(use_tc_tiling_on_sc=False),
        scratch_types=[
            pltpu.VMEM((CW, K), jnp.int32),
            pltpu.VMEM((CW, K), jnp.int32),
            pltpu.VMEM((NB, K, H), jnp.float32),
            pltpu.VMEM_SHARED((NPAD, H), jnp.float32),
            pltpu.SemaphoreType.DMA((NB,)),
            pltpu.SemaphoreType.DMA((NB,)),
        ],
    )


_B = 1000
_GRID = N // _B


def _norms_body(degs_ref, out_ref):
    d = degs_ref[...]
    dsrc = d[0, 0] + d[1, 0]
    ddst = d[0, 1] + d[1, 1]
    nsrc = lax.rsqrt(jnp.maximum(dsrc[:, :1], 1.0))
    ndst = lax.rsqrt(jnp.maximum(ddst[:, :1], 1.0))
    out_ref[0] = jnp.broadcast_to(nsrc, (_B, H))
    out_ref[1] = jnp.broadcast_to(ndst, (_B, H))


def _tc_norms(degs):
    return pl.pallas_call(
        _norms_body,
        grid=(_GRID,),
        in_specs=[pl.BlockSpec((NC, 2, _B, 16), lambda i: (0, 0, i, 0))],
        out_specs=pl.BlockSpec((2, _B, H), lambda i: (0, i, 0)),
        out_shape=jax.ShapeDtypeStruct((2, N, H), jnp.float32),
    )(degs)


def _first_body(x_ref, norms_ref, w_ref, out_ref):
    out_ref[...] = jnp.dot(x_ref[...], w_ref[...],
                           preferred_element_type=jnp.float32) * norms_ref[0]


def _tc_first(x, norms, W0):
    return pl.pallas_call(
        _first_body,
        grid=(_GRID,),
        in_specs=[
            pl.BlockSpec((_B, D_IN), lambda i: (i, 0)),
            pl.BlockSpec((2, _B, H), lambda i: (0, i, 0)),
            pl.BlockSpec((D_IN, H), lambda i: (0, 0)),
        ],
        out_specs=pl.BlockSpec((_B, H), lambda i: (i, 0)),
        out_shape=jax.ShapeDtypeStruct((N, H), jnp.float32),
    )(x, norms, W0)


def _mid_body(agg_ref, norms_ref, b_ref, w_ref, out_ref):
    agg = agg_ref[0] + agg_ref[1]
    h = agg * norms_ref[1] + b_ref[...]
    h = jnp.where(h >= 0.0, h, 0.01 * h)
    out_ref[...] = jnp.dot(h * norms_ref[0], w_ref[...],
                           preferred_element_type=jnp.float32)


def _tc_mid(agg, norms, b, W):
    return pl.pallas_call(
        _mid_body,
        grid=(_GRID,),
        in_specs=[
            pl.BlockSpec((NC, _B, H), lambda i: (0, i, 0)),
            pl.BlockSpec((2, _B, H), lambda i: (0, i, 0)),
            pl.BlockSpec((1, H), lambda i: (0, 0)),
            pl.BlockSpec((H, H), lambda i: (0, 0)),
        ],
        out_specs=pl.BlockSpec((_B, H), lambda i: (i, 0)),
        out_shape=jax.ShapeDtypeStruct((N, H), jnp.float32),
    )(agg, norms, b, W)


def _final_body(agg_ref, norms_ref, b_ref, fcw_ref, fcb_ref, out_ref):
    agg = agg_ref[0] + agg_ref[1]
    h = agg * norms_ref[1] + b_ref[...]
    h = jnp.where(h >= 0.0, h, 0.01 * h)
    out_ref[...] = (jnp.dot(h, fcw_ref[...], preferred_element_type=jnp.float32)
                    + fcb_ref[...])


def _tc_final(agg, norms, b, fcW, fcb):
    return pl.pallas_call(
        _final_body,
        grid=(_GRID,),
        in_specs=[
            pl.BlockSpec((NC, _B, H), lambda i: (0, i, 0)),
            pl.BlockSpec((2, _B, H), lambda i: (0, i, 0)),
            pl.BlockSpec((1, H), lambda i: (0, 0)),
            pl.BlockSpec((H, C), lambda i: (0, 0)),
            pl.BlockSpec((1, C), lambda i: (0, 0)),
        ],
        out_specs=pl.BlockSpec((_B, C), lambda i: (i, 0)),
        out_shape=jax.ShapeDtypeStruct((N, C), jnp.float32),
    )(agg, norms, b, fcW, fcb)


def kernel(x, edge_index, Ws, bs, fcW, fcb):
    src3d = edge_index[0].reshape(NW, CW, K)
    dst3d = edge_index[1].reshape(NW, CW, K)
    zeros16 = jnp.zeros((NS, RPT, 16), jnp.float32)
    zeros64 = jnp.zeros((NS, RPT, H), jnp.float32)
    ones80 = jnp.ones((K, 16), jnp.float32)

    sc_degrees = _sc_degrees_kernel()
    sc_layer = _sc_layer_kernel()

    degs = sc_degrees(src3d, dst3d, zeros16, ones80)
    degs = degs.reshape(NC, 2, NPAD, 16)[:, :, :N]
    norms = _tc_norms(degs)

    hw = _tc_first(x, norms, Ws[0])
    for l in range(1, len(Ws)):
        agg = sc_layer(hw, src3d, dst3d, zeros64)
        agg = agg.reshape(NC, NPAD, H)[:, :N]
        hw = _tc_mid(agg, norms, bs[l - 1].reshape(1, H), Ws[l])
    agg = sc_layer(hw, src3d, dst3d, zeros64)
    agg = agg.reshape(NC, NPAD, H)[:, :N]
    return _tc_final(agg, norms, bs[-1].reshape(1, H), fcW, fcb.reshape(1, C))


# speedup vs baseline: 15.6519x; 1.0845x over previous
import functools
import jax
import jax.numpy as jnp
from jax import lax
from jax.experimental import pallas as pl
from jax.experimental.pallas import tpu as pltpu
from jax.experimental.pallas import tpu_sc as plsc

N = 10000
E = 320000
D_IN = 128
H = 64
C = 40

NC = 2
NS = 16
NW = NC * NS
K = 80
CHUNKS = E // K
CW = CHUNKS // NW
RPT = 640
NPAD = NS * RPT


def _deg_body(src_hbm, dst_hbm, zeros_hbm, ones_hbm, out_hbm,
              idx_s, idx_d, ones_v, deg_s, deg_d):
    c = lax.axis_index("c")
    s = lax.axis_index("s")
    w = c * NS + s

    pltpu.sync_copy(ones_hbm, ones_v)
    pltpu.sync_copy(src_hbm.at[w], idx_s)
    pltpu.sync_copy(dst_hbm.at[w], idx_d)

    rbase = s * RPT
    pltpu.sync_copy(zeros_hbm.at[s], deg_s.at[pl.ds(rbase, RPT)])
    pltpu.sync_copy(zeros_hbm.at[s], deg_d.at[pl.ds(rbase, RPT)])
    plsc.subcore_barrier()

    def body(j, carry):
        pltpu.sync_copy(ones_v, deg_s.at[idx_s.at[j]], add=True)
        pltpu.sync_copy(ones_v, deg_d.at[idx_d.at[j]], add=True)
        return carry

    lax.fori_loop(0, CW, body, 0)
    plsc.subcore_barrier()

    pltpu.sync_copy(deg_s.at[pl.ds(rbase, RPT)], out_hbm.at[c, 0, s])
    pltpu.sync_copy(deg_d.at[pl.ds(rbase, RPT)], out_hbm.at[c, 1, s])


@functools.lru_cache(maxsize=None)
def _sc_degrees_kernel():
    mesh = plsc.VectorSubcoreMesh(core_axis_name="c", subcore_axis_name="s",
                                  num_cores=NC, num_subcores=NS)
    return pl.kernel(
        _deg_body,
        out_type=jax.ShapeDtypeStruct((NC, 2, NS, RPT, 16), jnp.float32),
        mesh=mesh,
        compiler_params=pltpu.CompilerParams(use_tc_tiling_on_sc=False),
        scratch_types=[
            pltpu.VMEM((CW, K), jnp.int32),
            pltpu.VMEM((CW, K), jnp.int32),
            pltpu.VMEM((K, 16), jnp.float32),
            pltpu.VMEM_SHARED((NPAD, 16), jnp.float32),
            pltpu.VMEM_SHARED((NPAD, 16), jnp.float32),
        ],
    )


NB = 5
NG = CW // NB


def _layer_body(hw_hbm, src_hbm, dst_hbm, zeros_hbm, out_hbm,
                idx_s, idx_d, msgs, agg_sh, gsems, ssems):
    c = lax.axis_index("c")
    s = lax.axis_index("s")
    w = c * NS + s

    pltpu.sync_copy(src_hbm.at[w], idx_s)
    pltpu.sync_copy(dst_hbm.at[w], idx_d)

    rbase = s * RPT
    pltpu.sync_copy(zeros_hbm.at[s], agg_sh.at[pl.ds(rbase, RPT)])
    plsc.subcore_barrier()

    dummy = hw_hbm.at[pl.ds(0, K)]

    for b in range(NB):
        pltpu.async_copy(hw_hbm.at[idx_s.at[b]], msgs.at[b], gsems.at[b])

    def group(g, carry):
        for b in range(NB):
            pltpu.make_async_copy(dummy, msgs.at[b], gsems.at[b]).wait()
            pltpu.async_copy(msgs.at[b], agg_sh.at[idx_d.at[g * NB + b]],
                             ssems.at[b], add=True)

        @pl.when(g < NG - 1)
        def _():
            for b in range(NB):
                pltpu.make_async_copy(dummy, msgs.at[b], ssems.at[b]).wait()
                pltpu.async_copy(hw_hbm.at[idx_s.at[(g + 1) * NB + b]],
                                 msgs.at[b], gsems.at[b])
        return carry

    lax.fori_loop(0, NG, group, 0)
    for b in range(NB):
        pltpu.make_async_copy(dummy, msgs.at[b], ssems.at[b]).wait()
    plsc.subcore_barrier()

    pltpu.sync_copy(agg_sh.at[pl.ds(rbase, RPT)], out_hbm.at[c, s])


@functools.lru_cache(maxsize=None)
def _sc_layer_kernel():
    mesh = plsc.VectorSubcoreMesh(core_axis_name="c", subcore_axis_name="s",
                                  num_cores=NC, num_subcores=NS)
    return pl.kernel(
        _layer_body,
        out_type=jax.ShapeDtypeStruct((NC, NS, RPT, H), jnp.float32),
        mesh=mesh,
        compiler_params=pltpu.CompilerParams(use_tc_tiling_on_sc=False),
        scratch_types=[
            pltpu.VMEM((CW, K), jnp.int32),
            pltpu.VMEM((CW, K), jnp.int32),
            pltpu.VMEM((NB, K, H), jnp.float32),
            pltpu.VMEM_SHARED((NPAD, H), jnp.float32),
            pltpu.SemaphoreType.DMA((NB,)),
            pltpu.SemaphoreType.DMA((NB,)),
        ],
    )


_B = 1024
_GRID = NPAD // _B


def _norms_body(degs_ref, out_ref):
    d = degs_ref[...]
    dsrc = d[0, 0] + d[1, 0]
    ddst = d[0, 1] + d[1, 1]
    nsrc = lax.rsqrt(jnp.maximum(dsrc[:, :1], 1.0))
    ndst = lax.rsqrt(jnp.maximum(ddst[:, :1], 1.0))
    out_ref[0] = jnp.broadcast_to(nsrc, (_B, H))
    out_ref[1] = jnp.broadcast_to(ndst, (_B, H))


def _tc_norms(degs):
    return pl.pallas_call(
        _norms_body,
        grid=(_GRID,),
        in_specs=[pl.BlockSpec((NC, 2, _B, 16), lambda i: (0, 0, i, 0))],
        out_specs=pl.BlockSpec((2, _B, H), lambda i: (0, i, 0)),
        out_shape=jax.ShapeDtypeStruct((2, NPAD, H), jnp.float32),
    )(degs)


def _first_body(x_ref, norms_ref, w_ref, out_ref):
    out_ref[...] = jnp.dot(x_ref[...], w_ref[...],
                           preferred_element_type=jnp.float32) * norms_ref[0]


def _tc_first(x, norms, W0):
    return pl.pallas_call(
        _first_body,
        grid=(_GRID,),
        in_specs=[
            pl.BlockSpec((_B, D_IN), lambda i: (i, 0)),
            pl.BlockSpec((2, _B, H), lambda i: (0, i, 0)),
            pl.BlockSpec((D_IN, H), lambda i: (0, 0)),
        ],
        out_specs=pl.BlockSpec((_B, H), lambda i: (i, 0)),
        out_shape=jax.ShapeDtypeStruct((NPAD, H), jnp.float32),
    )(x, norms, W0)


def _mid_body(agg_ref, norms_ref, b_ref, w_ref, out_ref):
    agg = agg_ref[0] + agg_ref[1]
    h = agg * norms_ref[1] + b_ref[...]
    h = jnp.where(h >= 0.0, h, 0.01 * h)
    out_ref[...] = jnp.dot(h * norms_ref[0], w_ref[...],
                           preferred_element_type=jnp.float32)


def _tc_mid(agg, norms, b, W):
    return pl.pallas_call(
        _mid_body,
        grid=(_GRID,),
        in_specs=[
            pl.BlockSpec((NC, _B, H), lambda i: (0, i, 0)),
            pl.BlockSpec((2, _B, H), lambda i: (0, i, 0)),
            pl.BlockSpec((1, H), lambda i: (0, 0)),
            pl.BlockSpec((H, H), lambda i: (0, 0)),
        ],
        out_specs=pl.BlockSpec((_B, H), lambda i: (i, 0)),
        out_shape=jax.ShapeDtypeStruct((NPAD, H), jnp.float32),
    )(agg, norms, b, W)


def _final_body(agg_ref, norms_ref, b_ref, fcw_ref, fcb_ref, out_ref):
    agg = agg_ref[0] + agg_ref[1]
    h = agg * norms_ref[1] + b_ref[...]
    h = jnp.where(h >= 0.0, h, 0.01 * h)
    out_ref[...] = (jnp.dot(h, fcw_ref[...], preferred_element_type=jnp.float32)
                    + fcb_ref[...])


def _tc_final(agg, norms, b, fcW, fcb):
    return pl.pallas_call(
        _final_body,
        grid=(_GRID,),
        in_specs=[
            pl.BlockSpec((NC, _B, H), lambda i: (0, i, 0)),
            pl.BlockSpec((2, _B, H), lambda i: (0, i, 0)),
            pl.BlockSpec((1, H), lambda i: (0, 0)),
            pl.BlockSpec((H, C), lambda i: (0, 0)),
            pl.BlockSpec((1, C), lambda i: (0, 0)),
        ],
        out_specs=pl.BlockSpec((_B, C), lambda i: (i, 0)),
        out_shape=jax.ShapeDtypeStruct((NPAD, C), jnp.float32),
    )(agg, norms, b, fcW, fcb)


def kernel(x, edge_index, Ws, bs, fcW, fcb):
    src3d = edge_index[0].reshape(NW, CW, K)
    dst3d = edge_index[1].reshape(NW, CW, K)
    zeros16 = jnp.zeros((NS, RPT, 16), jnp.float32)
    zeros64 = jnp.zeros((NS, RPT, H), jnp.float32)
    ones80 = jnp.ones((K, 16), jnp.float32)
    xp = jnp.pad(x, ((0, NPAD - N), (0, 0)))

    sc_degrees = _sc_degrees_kernel()
    sc_layer = _sc_layer_kernel()

    degs = sc_degrees(src3d, dst3d, zeros16, ones80)
    norms = _tc_norms(degs.reshape(NC, 2, NPAD, 16))

    hw = _tc_first(xp, norms, Ws[0])
    for l in range(1, len(Ws)):
        agg = sc_layer(hw, src3d, dst3d, zeros64)
        hw = _tc_mid(agg.reshape(NC, NPAD, H), norms,
                     bs[l - 1].reshape(1, H), Ws[l])
    agg = sc_layer(hw, src3d, dst3d, zeros64)
    out = _tc_final(agg.reshape(NC, NPAD, H), norms,
                    bs[-1].reshape(1, H), fcW, fcb.reshape(1, C))
    return out[:N]


# speedup vs baseline: 16.2715x; 1.0396x over previous
import functools
import jax
import jax.numpy as jnp
from jax import lax
from jax.experimental import pallas as pl
from jax.experimental.pallas import tpu as pltpu
from jax.experimental.pallas import tpu_sc as plsc

N = 10000
E = 320000
D_IN = 128
H = 64
C = 40

NC = 2
NS = 16
NW = NC * NS
K = 80
CHUNKS = E // K
CW = CHUNKS // NW
RPT = 640
NPAD = NS * RPT


def _deg_body(src_hbm, dst_hbm, zeros_hbm, ones_hbm, out_hbm,
              idx_s, idx_d, ones_v, deg_s, deg_d, dsem):
    c = lax.axis_index("c")
    s = lax.axis_index("s")
    w = c * NS + s

    pltpu.sync_copy(ones_hbm, ones_v)
    pltpu.sync_copy(src_hbm.at[w], idx_s)
    pltpu.sync_copy(dst_hbm.at[w], idx_d)

    rbase = s * RPT
    pltpu.sync_copy(zeros_hbm.at[s], deg_s.at[pl.ds(rbase, RPT)])
    pltpu.sync_copy(zeros_hbm.at[s], deg_d.at[pl.ds(rbase, RPT)])
    plsc.subcore_barrier()

    dummy = ones_hbm

    def body(j, carry):
        pltpu.async_copy(ones_v, deg_s.at[idx_s.at[j]], dsem, add=True)
        pltpu.async_copy(ones_v, deg_d.at[idx_d.at[j]], dsem, add=True)

        @pl.when(j >= 4)
        def _():
            pltpu.make_async_copy(dummy, ones_v, dsem).wait()
            pltpu.make_async_copy(dummy, ones_v, dsem).wait()
        return carry

    lax.fori_loop(0, CW, body, 0)
    for _ in range(8):
        pltpu.make_async_copy(dummy, ones_v, dsem).wait()
    plsc.subcore_barrier()

    pltpu.sync_copy(deg_s.at[pl.ds(rbase, RPT)], out_hbm.at[c, 0, s])
    pltpu.sync_copy(deg_d.at[pl.ds(rbase, RPT)], out_hbm.at[c, 1, s])


@functools.lru_cache(maxsize=None)
def _sc_degrees_kernel():
    mesh = plsc.VectorSubcoreMesh(core_axis_name="c", subcore_axis_name="s",
                                  num_cores=NC, num_subcores=NS)
    return pl.kernel(
        _deg_body,
        out_type=jax.ShapeDtypeStruct((NC, 2, NS, RPT, 16), jnp.float32),
        mesh=mesh,
        compiler_params=pltpu.CompilerParams(use_tc_tiling_on_sc=False),
        scratch_types=[
            pltpu.VMEM((CW, K), jnp.int32),
            pltpu.VMEM((CW, K), jnp.int32),
            pltpu.VMEM((K, 16), jnp.float32),
            pltpu.VMEM_SHARED((NPAD, 16), jnp.float32),
            pltpu.VMEM_SHARED((NPAD, 16), jnp.float32),
            pltpu.SemaphoreType.DMA,
        ],
    )


NB = 5
NG = CW // NB


def _layer_body(hw_hbm, src_hbm, dst_hbm, zeros_hbm, out_hbm,
                idx_s, idx_d, msgs, agg_sh, gsems, ssems):
    c = lax.axis_index("c")
    s = lax.axis_index("s")
    w = c * NS + s

    pltpu.sync_copy(src_hbm.at[w], idx_s)
    pltpu.sync_copy(dst_hbm.at[w], idx_d)

    rbase = s * RPT
    pltpu.sync_copy(zeros_hbm.at[s], agg_sh.at[pl.ds(rbase, RPT)])
    plsc.subcore_barrier()

    dummy = hw_hbm.at[pl.ds(0, K)]

    def wait_g(b):
        pltpu.make_async_copy(dummy, msgs.at[b], gsems.at[b]).wait()

    def wait_s(b):
        pltpu.make_async_copy(dummy, msgs.at[b], ssems.at[b]).wait()

    def fire_g(grp, b):
        pltpu.async_copy(hw_hbm.at[idx_s.at[grp * NB + (b % NB)]],
                         msgs.at[b], gsems.at[b])

    def fire_s(grp, b):
        pltpu.async_copy(msgs.at[b], agg_sh.at[idx_d.at[grp * NB + (b % NB)]],
                         ssems.at[b], add=True)

    for b in range(NB):
        fire_g(0, b)
    for b in range(NB):
        fire_g(1, NB + b)

    def super_group(t, carry):
        g0 = 2 * t
        for b in range(NB):
            wait_g(b)
            fire_s(g0, b)
        for b in range(NB):
            wait_g(NB + b)
            fire_s(g0 + 1, NB + b)

        @pl.when(g0 + 2 < NG)
        def _():
            for b in range(NB):
                wait_s(b)
                fire_g(g0 + 2, b)

        @pl.when(g0 + 3 < NG)
        def _():
            for b in range(NB):
                wait_s(NB + b)
                fire_g(g0 + 3, NB + b)
        return carry

    lax.fori_loop(0, NG // 2, super_group, 0)
    for b in range(NB):
        wait_g(b)
        fire_s(NG - 1, b)
    for b in range(2 * NB):
        wait_s(b)
    plsc.subcore_barrier()

    pltpu.sync_copy(agg_sh.at[pl.ds(rbase, RPT)], out_hbm.at[c, s])


@functools.lru_cache(maxsize=None)
def _sc_layer_kernel():
    mesh = plsc.VectorSubcoreMesh(core_axis_name="c", subcore_axis_name="s",
                                  num_cores=NC, num_subcores=NS)
    return pl.kernel(
        _layer_body,
        out_type=jax.ShapeDtypeStruct((NC, NS, RPT, H), jnp.float32),
        mesh=mesh,
        compiler_params=pltpu.CompilerParams(use_tc_tiling_on_sc=False),
        scratch_types=[
            pltpu.VMEM((CW, K), jnp.int32),
            pltpu.VMEM((CW, K), jnp.int32),
            pltpu.VMEM((2 * NB, K, H), jnp.float32),
            pltpu.VMEM_SHARED((NPAD, H), jnp.float32),
            pltpu.SemaphoreType.DMA((2 * NB,)),
            pltpu.SemaphoreType.DMA((2 * NB,)),
        ],
    )


_B = 1024
_GRID = NPAD // _B


def _norms_body(degs_ref, out_ref):
    d = degs_ref[...]
    dsrc = d[0, 0] + d[1, 0]
    ddst = d[0, 1] + d[1, 1]
    nsrc = lax.rsqrt(jnp.maximum(dsrc[:, :1], 1.0))
    ndst = lax.rsqrt(jnp.maximum(ddst[:, :1], 1.0))
    out_ref[0] = jnp.broadcast_to(nsrc, (_B, H))
    out_ref[1] = jnp.broadcast_to(ndst, (_B, H))


def _tc_norms(degs):
    return pl.pallas_call(
        _norms_body,
        grid=(_GRID,),
        in_specs=[pl.BlockSpec((NC, 2, _B, 16), lambda i: (0, 0, i, 0))],
        out_specs=pl.BlockSpec((2, _B, H), lambda i: (0, i, 0)),
        out_shape=jax.ShapeDtypeStruct((2, NPAD, H), jnp.float32),
    )(degs)


def _first_body(x_ref, norms_ref, w_ref, out_ref):
    out_ref[...] = jnp.dot(x_ref[...], w_ref[...],
                           preferred_element_type=jnp.float32) * norms_ref[0]


def _tc_first(x, norms, W0):
    return pl.pallas_call(
        _first_body,
        grid=(_GRID,),
        in_specs=[
            pl.BlockSpec((_B, D_IN), lambda i: (i, 0)),
            pl.BlockSpec((2, _B, H), lambda i: (0, i, 0)),
            pl.BlockSpec((D_IN, H), lambda i: (0, 0)),
        ],
        out_specs=pl.BlockSpec((_B, H), lambda i: (i, 0)),
        out_shape=jax.ShapeDtypeStruct((NPAD, H), jnp.float32),
    )(x, norms, W0)


def _mid_body(agg_ref, norms_ref, b_ref, w_ref, out_ref):
    agg = agg_ref[0] + agg_ref[1]
    h = agg * norms_ref[1] + b_ref[...]
    h = jnp.where(h >= 0.0, h, 0.01 * h)
    out_ref[...] = jnp.dot(h * norms_ref[0], w_ref[...],
                           preferred_element_type=jnp.float32)


def _tc_mid(agg, norms, b, W):
    return pl.pallas_call(
        _mid_body,
        grid=(_GRID,),
        in_specs=[
            pl.BlockSpec((NC, _B, H), lambda i: (0, i, 0)),
            pl.BlockSpec((2, _B, H), lambda i: (0, i, 0)),
            pl.BlockSpec((1, H), lambda i: (0, 0)),
            pl.BlockSpec((H, H), lambda i: (0, 0)),
        ],
        out_specs=pl.BlockSpec((_B, H), lambda i: (i, 0)),
        out_shape=jax.ShapeDtypeStruct((NPAD, H), jnp.float32),
    )(agg, norms, b, W)


def _final_body(agg_ref, norms_ref, b_ref, fcw_ref, fcb_ref, out_ref):
    agg = agg_ref[0] + agg_ref[1]
    h = agg * norms_ref[1] + b_ref[...]
    h = jnp.where(h >= 0.0, h, 0.01 * h)
    out_ref[...] = (jnp.dot(h, fcw_ref[...], preferred_element_type=jnp.float32)
                    + fcb_ref[...])


def _tc_final(agg, norms, b, fcW, fcb):
    return pl.pallas_call(
        _final_body,
        grid=(_GRID,),
        in_specs=[
            pl.BlockSpec((NC, _B, H), lambda i: (0, i, 0)),
            pl.BlockSpec((2, _B, H), lambda i: (0, i, 0)),
            pl.BlockSpec((1, H), lambda i: (0, 0)),
            pl.BlockSpec((H, C), lambda i: (0, 0)),
            pl.BlockSpec((1, C), lambda i: (0, 0)),
        ],
        out_specs=pl.BlockSpec((_B, C), lambda i: (i, 0)),
        out_shape=jax.ShapeDtypeStruct((NPAD, C), jnp.float32),
    )(agg, norms, b, fcW, fcb)


def kernel(x, edge_index, Ws, bs, fcW, fcb):
    src3d = edge_index[0].reshape(NW, CW, K)
    dst3d = edge_index[1].reshape(NW, CW, K)
    zeros16 = jnp.zeros((NS, RPT, 16), jnp.float32)
    zeros64 = jnp.zeros((NS, RPT, H), jnp.float32)
    ones80 = jnp.ones((K, 16), jnp.float32)
    xp = jnp.pad(x, ((0, NPAD - N), (0, 0)))

    sc_degrees = _sc_degrees_kernel()
    sc_layer = _sc_layer_kernel()

    degs = sc_degrees(src3d, dst3d, zeros16, ones80)
    norms = _tc_norms(degs.reshape(NC, 2, NPAD, 16))

    hw = _tc_first(xp, norms, Ws[0])
    for l in range(1, len(Ws)):
        agg = sc_layer(hw, src3d, dst3d, zeros64)
        hw = _tc_mid(agg.reshape(NC, NPAD, H), norms,
                     bs[l - 1].reshape(1, H), Ws[l])
    agg = sc_layer(hw, src3d, dst3d, zeros64)
    out = _tc_final(agg.reshape(NC, NPAD, H), norms,
                    bs[-1].reshape(1, H), fcW, fcb.reshape(1, C))
    return out[:N]
